# baseline (speedup 1.0000x reference)
import functools

import jax
import jax.numpy as jnp
from jax import lax
from jax.experimental import pallas as pl

NUM_TOKENS = 1024
LATENT = 64
SPOS = 1024
CHUNK = 256
BPS = 4


def _vq_block(xb_ref, cb_ref, csq_ref, cbhit_ref, codes_ref, idx_ref):
    cb = cb_ref[...]
    csq = csq_ref[...]
    cbhit = cbhit_ref[...]
    for bb in range(BPS):
        for j in range(SPOS // CHUNK):
            cols = pl.ds(j * CHUNK, CHUNK)
            xb = xb_ref[bb, :, cols]
            prod = lax.dot_general(xb, cb, (((0,), (1,)), ((), ())),
                                   preferred_element_type=jnp.float32)
            xsq = jnp.sum(xb * xb, axis=0)[:, None]
            dist = (xsq - 2.0 * prod) + csq
            idx = jnp.argmin(dist, axis=1).astype(jnp.int32)
            onehot_t = (lax.broadcasted_iota(jnp.int32, (NUM_TOKENS, CHUNK), 0)
                        == idx[None, :]).astype(jnp.bfloat16)
            codes_ref[bb, :, cols] = lax.dot_general(
                cbhit, onehot_t, (((1,), (0,)), ((), ())),
                preferred_element_type=jnp.float32)
            idx_ref[bb, 0, cols] = idx


@functools.partial(jax.jit, static_argnames=("interpret",))
def kernel(x, codebook, interpret=False):
    b, c, h, w = x.shape
    hw = h * w
    xr = x.reshape(b, c, hw)
    csq = jnp.sum(codebook * codebook, axis=-1)[None, :]
    cb_hi = codebook.astype(jnp.bfloat16)
    codes_r, idx_r = pl.pallas_call(
        _vq_block,
        grid=(b // BPS,),
        in_specs=[
            pl.BlockSpec((BPS, LATENT, hw), lambda i: (i, 0, 0)),
            pl.BlockSpec((NUM_TOKENS, LATENT), lambda i: (0, 0)),
            pl.BlockSpec((1, NUM_TOKENS), lambda i: (0, 0)),
            pl.BlockSpec((LATENT, NUM_TOKENS), lambda i: (0, 0)),
        ],
        out_specs=[
            pl.BlockSpec((BPS, LATENT, hw), lambda i: (i, 0, 0)),
            pl.BlockSpec((BPS, 1, hw), lambda i: (i, 0, 0)),
        ],
        out_shape=[
            jax.ShapeDtypeStruct((b, LATENT, hw), jnp.float32),
            jax.ShapeDtypeStruct((b, 1, hw), jnp.int32),
        ],
        interpret=interpret,
    )(xr, codebook, csq, cb_hi.T)
    codes = codes_r.reshape(b, c, h, w)
    indices = idx_r.reshape(b, h, w)
    return (codes, indices)


# speedup vs baseline: 1.0386x; 1.0386x over previous
import functools

import jax
import jax.numpy as jnp
from jax import lax
from jax.experimental import pallas as pl

NUM_TOKENS = 1024
LATENT = 64
SPOS = 1024
CHUNK = 256
BPS = 4


def _vq_block(xb_ref, cbm2_ref, csq_ref, cbhit_ref, cblot_ref, codes_ref, idx_ref):
    cbm2 = cbm2_ref[...]
    csq = csq_ref[...]
    cbhit = cbhit_ref[...]
    cblot = cblot_ref[...]
    for bb in range(BPS):
        for j in range(SPOS // CHUNK):
            cols = pl.ds(j * CHUNK, CHUNK)
            xb = xb_ref[bb, :, cols]
            prodm2 = lax.dot_general(xb, cbm2, (((0,), (1,)), ((), ())),
                                     preferred_element_type=jnp.float32)
            xsq = jnp.sum(xb * xb, axis=0)[:, None]
            dist = (xsq + prodm2) + csq
            idx = jnp.argmin(dist, axis=1).astype(jnp.int32)
            onehot_t = (lax.broadcasted_iota(jnp.int32, (NUM_TOKENS, CHUNK), 0)
                        == idx[None, :]).astype(jnp.bfloat16)
            hi = lax.dot_general(cbhit, onehot_t, (((1,), (0,)), ((), ())),
                                 preferred_element_type=jnp.float32)
            lo = lax.dot_general(cblot, onehot_t, (((1,), (0,)), ((), ())),
                                 preferred_element_type=jnp.float32)
            codes_ref[bb, :, cols] = hi
            codes_ref[bb, :, cols] += lo
            idx_ref[bb, 0, cols] = idx


@functools.partial(jax.jit, static_argnames=("interpret",))
def kernel(x, codebook, interpret=False):
    b, c, h, w = x.shape
    hw = h * w
    xr = x.reshape(b, c, hw)
    csq = jnp.sum(codebook * codebook, axis=-1)[None, :]
    cbm2 = -2.0 * codebook
    cb_u = lax.bitcast_convert_type(codebook, jnp.uint32)
    cb_hi = lax.bitcast_convert_type(
        (cb_u >> 16).astype(jnp.uint16), jnp.bfloat16)
    cb_lo = (codebook - cb_hi.astype(jnp.float32)).astype(jnp.bfloat16)
    codes_r, idx_r = pl.pallas_call(
        _vq_block,
        grid=(b // BPS,),
        in_specs=[
            pl.BlockSpec((BPS, LATENT, hw), lambda i: (i, 0, 0)),
            pl.BlockSpec((NUM_TOKENS, LATENT), lambda i: (0, 0)),
            pl.BlockSpec((1, NUM_TOKENS), lambda i: (0, 0)),
            pl.BlockSpec((LATENT, NUM_TOKENS), lambda i: (0, 0)),
            pl.BlockSpec((LATENT, NUM_TOKENS), lambda i: (0, 0)),
        ],
        out_specs=[
            pl.BlockSpec((BPS, LATENT, hw), lambda i: (i, 0, 0)),
            pl.BlockSpec((BPS, 1, hw), lambda i: (i, 0, 0)),
        ],
        out_shape=[
            jax.ShapeDtypeStruct((b, LATENT, hw), jnp.float32),
            jax.ShapeDtypeStruct((b, 1, hw), jnp.int32),
        ],
        interpret=interpret,
    )(xr, cbm2, csq, cb_hi.T, cb_lo.T)
    codes = codes_r.reshape(b, c, h, w)
    indices = idx_r.reshape(b, h, w)
    return (codes, indices)
